# baseline (speedup 1.0000x reference)
import functools

import jax
import jax.numpy as jnp
from jax import lax
from jax.experimental import pallas as pl
from jax.experimental.pallas import tpu as pltpu
from jax.experimental.pallas import tpu_sc as plsc

_VOCAB = 1000000
_D = 64
_T = 16384
_SCALE = 8.0

_NC = 2
_NS = 16
_NW = _NC * _NS
_RPW = _T // _NW
_S = 4


def _sc_kernel(idx_hbm, table_t_hbm, out_t_hbm, idx_v, flat_v, *rest):
    slots = rest[:_S]
    sems = rest[_S:2 * _S]
    osem = rest[2 * _S]

    wid = lax.axis_index("s") * _NC + lax.axis_index("c")
    base = pl.multiple_of(wid * _RPW, 128)
    pltpu.sync_copy(idx_hbm.at[pl.ds(base, _RPW)], idx_v)

    lane = lax.iota(jnp.int32, 16)

    def token_id(t):
        w0 = pl.multiple_of((t // 16) * 16, 16)
        window = idx_v[pl.ds(w0, 16)]
        return jnp.sum(jnp.where(lane == t - w0, window, 0))

    def fire(t, slot, sem):
        r = token_id(t)
        q = pl.multiple_of((r // 128) * 128, 128)
        pltpu.make_async_copy(
            table_t_hbm.at[:, pl.ds(q, 128)], slot, sem
        ).start()
        return r - q

    def extract(t, slot, sem, c):
        pltpu.make_async_copy(
            table_t_hbm.at[:, pl.ds(0, 128)], slot, sem
        ).wait()
        cvec = jnp.full((16,), c, jnp.int32)
        for j in range(_D // 16):
            dvec = lane + (16 * j)
            vals = plsc.load_gather(slot, [dvec, cvec]) * _SCALE
            tgt = dvec * _RPW + t
            plsc.store_scatter(flat_v, [tgt], vals)

    cols = [fire(jnp.int32(s), slots[s], sems[s]) for s in range(_S)]

    def step(k, carry):
        i = k * _S
        out = []
        for s in range(_S):
            extract(i + s, slots[s], sems[s], carry[s])
            out.append(fire(i + _S + s, slots[s], sems[s]))
        return tuple(out)

    cols = lax.fori_loop(0, _RPW // _S - 1, step, tuple(cols))

    for s in range(_S):
        extract(jnp.int32(_RPW - _S + s), slots[s], sems[s], cols[s])

    def out_row(d, _):
        pltpu.make_async_copy(
            flat_v.at[pl.ds(d * _RPW, _RPW)],
            out_t_hbm.at[d, pl.ds(base, _RPW)],
            osem,
        ).start()
        return _

    lax.fori_loop(0, _D, out_row, 0)

    def out_drain(d, _):
        pltpu.make_async_copy(
            flat_v.at[pl.ds(0, _RPW)],
            out_t_hbm.at[0, pl.ds(base, _RPW)],
            osem,
        ).wait()
        return _

    lax.fori_loop(0, _D, out_drain, 0, unroll=8)


@jax.jit
def kernel(x, input_embedding_table_VD):
    idx = x.astype(jnp.int32)
    table_t = input_embedding_table_VD.T
    mesh = plsc.VectorSubcoreMesh(core_axis_name="c", subcore_axis_name="s")
    run = pl.kernel(
        _sc_kernel,
        mesh=mesh,
        out_type=jax.ShapeDtypeStruct((_D, _T), jnp.float32),
        scratch_types=[
            pltpu.VMEM((_RPW,), jnp.int32),
            pltpu.VMEM((_D * _RPW,), jnp.float32),
        ]
        + [pltpu.VMEM((_D, 128), jnp.float32) for _ in range(_S)]
        + [pltpu.SemaphoreType.DMA for _ in range(_S)]
        + [pltpu.SemaphoreType.DMA],
        compiler_params=pltpu.CompilerParams(
            needs_layout_passes=False, skip_device_barrier=True
        ),
    )
    out_t = run(idx, table_t)
    return out_t.T


# speedup vs baseline: 1.1782x; 1.1782x over previous
import functools

import jax
import jax.numpy as jnp
from jax import lax
from jax.experimental import pallas as pl
from jax.experimental.pallas import tpu as pltpu
from jax.experimental.pallas import tpu_sc as plsc

_VOCAB = 1000000
_D = 64
_T = 16384
_SCALE = 8.0

_NC = 2
_NS = 16
_NW = _NC * _NS
_RPW = _T // _NW
_S = 11
_NFULL = _RPW // _S
_TAIL = _RPW - _NFULL * _S


def _sc_kernel(idx_hbm, table_t_hbm, out_t_hbm, idx_v, flat_v, *rest):
    slots = rest[:_S]
    sems = rest[_S:2 * _S]
    osem = rest[2 * _S]

    wid = lax.axis_index("s") * _NC + lax.axis_index("c")
    base = pl.multiple_of(wid * _RPW, 128)
    pltpu.sync_copy(idx_hbm.at[pl.ds(base, _RPW)], idx_v)

    lane = lax.iota(jnp.int32, 16)

    def token_id(t):
        w0 = pl.multiple_of((t // 16) * 16, 16)
        window = idx_v[pl.ds(w0, 16)]
        return jnp.sum(jnp.where(lane == t - w0, window, 0))

    def fire(t, slot, sem):
        r = token_id(t)
        q = pl.multiple_of((r // 128) * 128, 128)
        pltpu.make_async_copy(
            table_t_hbm.at[:, pl.ds(q, 128)], slot, sem
        ).start()
        return r - q

    def extract(t, slot, sem, c):
        pltpu.make_async_copy(
            table_t_hbm.at[:, pl.ds(0, 128)], slot, sem
        ).wait()
        cvec = jnp.full((16,), c, jnp.int32)
        for j in range(_D // 16):
            dvec = lane + (16 * j)
            vals = plsc.load_gather(slot, [dvec, cvec]) * _SCALE
            tgt = dvec * _RPW + t
            plsc.store_scatter(flat_v, [tgt], vals)

    cols = [fire(jnp.int32(s), slots[s], sems[s]) for s in range(_S)]

    def step(k, carry):
        i = k * _S
        out = []
        for s in range(_S):
            extract(i + s, slots[s], sems[s], carry[s])
            out.append(fire(i + _S + s, slots[s], sems[s]))
        return tuple(out)

    cols = lax.fori_loop(0, _NFULL - 1, step, tuple(cols))

    for s in range(_S):
        extract(jnp.int32((_NFULL - 1) * _S + s), slots[s], sems[s], cols[s])

    tail_cols = [
        fire(jnp.int32(_NFULL * _S + s), slots[s], sems[s])
        for s in range(_TAIL)
    ]
    for s in range(_TAIL):
        extract(jnp.int32(_NFULL * _S + s), slots[s], sems[s], tail_cols[s])

    def out_row(d, _):
        pltpu.make_async_copy(
            flat_v.at[pl.ds(d * _RPW, _RPW)],
            out_t_hbm.at[d, pl.ds(base, _RPW)],
            osem,
        ).start()
        return _

    lax.fori_loop(0, _D, out_row, 0)

    def out_drain(d, _):
        pltpu.make_async_copy(
            flat_v.at[pl.ds(0, _RPW)],
            out_t_hbm.at[0, pl.ds(base, _RPW)],
            osem,
        ).wait()
        return _

    lax.fori_loop(0, _D, out_drain, 0, unroll=8)


@jax.jit
def kernel(x, input_embedding_table_VD):
    idx = x.astype(jnp.int32)
    table_t = input_embedding_table_VD.T
    mesh = plsc.VectorSubcoreMesh(core_axis_name="c", subcore_axis_name="s")
    run = pl.kernel(
        _sc_kernel,
        mesh=mesh,
        out_type=jax.ShapeDtypeStruct((_D, _T), jnp.float32),
        scratch_types=[
            pltpu.VMEM((_RPW,), jnp.int32),
            pltpu.VMEM((_D * _RPW,), jnp.float32),
        ]
        + [pltpu.VMEM((_D, 128), jnp.float32) for _ in range(_S)]
        + [pltpu.SemaphoreType.DMA for _ in range(_S)]
        + [pltpu.SemaphoreType.DMA],
        compiler_params=pltpu.CompilerParams(
            needs_layout_passes=False, skip_device_barrier=True
        ),
    )
    out_t = run(idx, table_t)
    return out_t.T
